# Initial kernel scaffold; baseline (speedup 1.0000x reference)
import jax
import jax.numpy as jnp
from jax.experimental import pallas as pl


def kernel(x, edge_index, batch, Wl1, bl1, Wr1, br1, att1, bias1, Wl2, bl2, Wr2, br2, att2, bias2, W3, b3, W4, b4):
    raise NotImplementedError("write your pallas kernel here")


# speedup vs baseline: 24.4214x; 24.4214x over previous
import jax
import jax.numpy as jnp
from jax import lax
from jax.experimental import pallas as pl
from jax.experimental.pallas import tpu as pltpu
from jax.experimental.pallas import tpu_sc as plsc

N = 10000
E_TOT = 170000
D_IN = 128
HID = 64
N_GRAPHS = 64

CHUNK = 128
SL1 = 10752
SL2 = 5376
E_PAD = 172032
NB = 10
BN = N // NB
STRIPE = N // 16
SUB = 125
AW = 72

_mesh = plsc.VectorSubcoreMesh(core_axis_name="c", subcore_axis_name="s",
                               num_cores=2, num_subcores=16)
_params = pltpu.CompilerParams(use_tc_tiling_on_sc=False,
                               needs_layout_passes=False)

_GDN = lax.GatherDimensionNumbers(offset_dims=(), collapsed_slice_dims=(0,),
                                  start_index_map=(0,))


def _shuffle(vec, idx):
    return lax.gather(vec, idx[:, None], _GDN, (1,),
                      mode=lax.GatherScatterMode.PROMISE_IN_BOUNDS)


def _tca_body(x_ref, wl_ref, bl_ref, wr_ref, br_ref, xl_ref, xr_ref):
    xb = x_ref[...]
    xl = jnp.dot(xb, wl_ref[...], preferred_element_type=jnp.float32) + bl_ref[...]
    xr = jnp.dot(xb, wr_ref[...], preferred_element_type=jnp.float32) + br_ref[...]
    for h in range(4):
        xl_ref[h] = xl[:, h * HID:(h + 1) * HID]
        xr_ref[h] = xr[:, h * HID:(h + 1) * HID]


def _tc_a(x, Wl1, bl1r, Wr1, br1r):
    return pl.pallas_call(
        _tca_body,
        grid=(NB,),
        in_specs=[
            pl.BlockSpec((BN, D_IN), lambda b: (b, 0)),
            pl.BlockSpec((D_IN, 4 * HID), lambda b: (0, 0)),
            pl.BlockSpec((1, 4 * HID), lambda b: (0, 0)),
            pl.BlockSpec((D_IN, 4 * HID), lambda b: (0, 0)),
            pl.BlockSpec((1, 4 * HID), lambda b: (0, 0)),
        ],
        out_specs=[
            pl.BlockSpec((4, BN, HID), lambda b: (0, b, 0)),
            pl.BlockSpec((4, BN, HID), lambda b: (0, b, 0)),
        ],
        out_shape=[
            jax.ShapeDtypeStruct((4, N, HID), jnp.float32),
            jax.ShapeDtypeStruct((4, N, HID), jnp.float32),
        ],
    )(x, Wl1, bl1r, Wr1, br1r)


def _edge_phase(head, stripe, n_chunks,
                src_hbm, dst_hbm, xl_hbm, xr_hbm, att_hbm,
                srcv, dstv, asrc, adst, xlr, xrr, prow, attv, acc_sh,
                sem1, sem2):
    pltpu.sync_copy(att_hbm.at[pl.ds(pl.multiple_of(head * HID, HID), HID)], attv)
    att_regs = [attv[pl.ds(k * 16, 16)] for k in range(4)]
    roff = jnp.full((16,), head * N, jnp.int32)
    lane = lax.iota(jnp.int32, 16)
    shift8 = (lane + 8) & 15

    def chunk_body(ci, _):
        base = stripe + ci * CHUNK
        pltpu.sync_copy(src_hbm.at[pl.ds(pl.multiple_of(base, CHUNK), CHUNK)], srcv)
        pltpu.sync_copy(dst_hbm.at[pl.ds(pl.multiple_of(base, CHUNK), CHUNK)], dstv)

        @plsc.parallel_loop(0, CHUNK // 16)
        def _adj(i):
            asrc[pl.ds(i * 16, 16)] = srcv[pl.ds(i * 16, 16)] + roff
            adst[pl.ds(i * 16, 16)] = dstv[pl.ds(i * 16, 16)] + roff

        cp1 = pltpu.async_copy(xl_hbm.at[asrc], xlr, sem1)
        cp2 = pltpu.async_copy(xr_hbm.at[adst], xrr, sem2)
        cp1.wait()
        cp2.wait()

        @plsc.parallel_loop(0, CHUNK, unroll=2)
        def _edge(e):
            zls = []
            acc = jnp.zeros((16,), jnp.float32)
            for cb in range(4):
                zl = xlr[e, pl.ds(cb * 16, 16)]
                zr = xrr[e, pl.ds(cb * 16, 16)]
                z = zl + zr
                z = jnp.maximum(z, 0.2 * z)
                acc = acc + z * att_regs[cb]
                zls.append(zl)
            logit = jnp.sum(acc)
            vf = (base + e < E_TOT).astype(jnp.float32)
            p = jnp.exp(jnp.full((16,), logit, jnp.float32)) * vf
            for cb in range(3):
                prow[e, pl.ds(cb * 16, 16)] = zls[cb] * p
            d3 = zls[3] * p
            prow[e, pl.ds(48, 16)] = d3
            tail = jnp.where(lane < 8, _shuffle(d3, shift8),
                             jnp.where(lane == 8, p, 0.0))
            prow[e, pl.ds(56, 16)] = tail

        pltpu.sync_copy(prow, acc_sh.at[dstv], add=True)
        return 0

    lax.fori_loop(0, n_chunks, chunk_body, 0)
    plsc.subcore_barrier()


def _zero_acc(nbuf, acc_sh, s):
    @plsc.parallel_loop(0, SUB)
    def _zrow(r):
        for k in range(AW // 16):
            nbuf[r, pl.ds(k * 16, 16)] = jnp.zeros((16,), jnp.float32)
        nbuf[r, pl.ds(AW - 16, 16)] = jnp.zeros((16,), jnp.float32)

    for k in range(5):
        pltpu.sync_copy(nbuf, acc_sh.at[pl.ds(s * STRIPE + k * SUB, SUB)])
    plsc.subcore_barrier()


def _make_scl1_body(q):
    def body(src_hbm, dst_hbm, xl_hbm, xr_hbm, att_hbm, out_hbm,
             srcv, dstv, asrc, adst, xlr, xrr, prow, attv, nbuf, obuf,
             acc_sh, sem1, sem2):
        c = lax.axis_index("c")
        s = lax.axis_index("s")
        _zero_acc(nbuf, acc_sh, s)
        head = 2 * q + c
        _edge_phase(head, s * SL1, SL1 // CHUNK,
                    src_hbm, dst_hbm, xl_hbm, xr_hbm, att_hbm,
                    srcv, dstv, asrc, adst, xlr, xrr, prow, attv, acc_sh,
                    sem1, sem2)
        idx8 = jnp.full((16,), 8, jnp.int32)
        for k in range(5):
            r0 = s * STRIPE + k * SUB
            pltpu.sync_copy(acc_sh.at[pl.ds(r0, SUB)], nbuf)

            @plsc.parallel_loop(0, SUB)
            def _nrow(r):
                svec = nbuf[r, pl.ds(56, 16)]
                inv = 1.0 / (svec + 1e-16)
                iv = _shuffle(inv, idx8)
                for cb in range(4):
                    obuf[r, pl.ds(cb * 16, 16)] = nbuf[r, pl.ds(cb * 16, 16)] * iv

            pltpu.sync_copy(obuf, out_hbm.at[pl.ds(head * N + r0, SUB)])

    return body


def _sc_l1(q, srcp, dstp, xl_tab, xr_tab, att_flat):
    return pl.kernel(
        _make_scl1_body(q),
        out_type=jax.ShapeDtypeStruct((4 * N, HID), jnp.float32),
        mesh=_mesh,
        compiler_params=_params,
        scratch_types=[
            pltpu.VMEM((CHUNK,), jnp.int32),
            pltpu.VMEM((CHUNK,), jnp.int32),
            pltpu.VMEM((CHUNK,), jnp.int32),
            pltpu.VMEM((CHUNK,), jnp.int32),
            pltpu.VMEM((CHUNK, HID), jnp.float32),
            pltpu.VMEM((CHUNK, HID), jnp.float32),
            pltpu.VMEM((CHUNK, AW), jnp.float32),
            pltpu.VMEM((HID,), jnp.float32),
            pltpu.VMEM((SUB, AW), jnp.float32),
            pltpu.VMEM((SUB, HID), jnp.float32),
            pltpu.VMEM_SHARED((N, AW), jnp.float32),
            pltpu.SemaphoreType.DMA,
            pltpu.SemaphoreType.DMA,
        ],
    )(srcp, dstp, xl_tab, xr_tab, att_flat)


def _tcb_body(h0_ref, h1_ref, h2_ref, h3_ref, b1_ref, wl_ref, bl_ref,
              wr_ref, br_ref, xl2_ref, xr2_ref):
    hs = [h0_ref, h1_ref, h2_ref, h3_ref]
    xl2 = bl_ref[...]
    xr2 = br_ref[...]
    for h in range(4):
        hh = jnp.maximum(hs[h][...] + b1_ref[h:h + 1, :], 0.0)
        xl2 = xl2 + jnp.dot(hh, wl_ref[pl.ds(h * HID, HID), :],
                            preferred_element_type=jnp.float32)
        xr2 = xr2 + jnp.dot(hh, wr_ref[pl.ds(h * HID, HID), :],
                            preferred_element_type=jnp.float32)
    xl2_ref[...] = xl2
    xr2_ref[...] = xr2


def _tc_b(o1a, o1b, b1r, Wl2, bl2r, Wr2, br2r):
    hspec = [
        pl.BlockSpec((BN, HID), lambda b: (0 * NB + b, 0)),
        pl.BlockSpec((BN, HID), lambda b: (1 * NB + b, 0)),
        pl.BlockSpec((BN, HID), lambda b: (2 * NB + b, 0)),
        pl.BlockSpec((BN, HID), lambda b: (3 * NB + b, 0)),
    ]
    return pl.pallas_call(
        _tcb_body,
        grid=(NB,),
        in_specs=hspec + [
            pl.BlockSpec((4, HID), lambda b: (0, 0)),
            pl.BlockSpec((4 * HID, HID), lambda b: (0, 0)),
            pl.BlockSpec((1, HID), lambda b: (0, 0)),
            pl.BlockSpec((4 * HID, HID), lambda b: (0, 0)),
            pl.BlockSpec((1, HID), lambda b: (0, 0)),
        ],
        out_specs=[
            pl.BlockSpec((BN, HID), lambda b: (b, 0)),
            pl.BlockSpec((BN, HID), lambda b: (b, 0)),
        ],
        out_shape=[
            jax.ShapeDtypeStruct((N, HID), jnp.float32),
            jax.ShapeDtypeStruct((N, HID), jnp.float32),
        ],
    )(o1a, o1a, o1b, o1b, b1r, Wl2, bl2r, Wr2, br2r)


def _scl2_body(src_hbm, dst_hbm, xl_hbm, xr_hbm, att_hbm, out_hbm,
               srcv, dstv, asrc, adst, xlr, xrr, prow, attv, nbuf,
               acc_sh, sem1, sem2):
    c = lax.axis_index("c")
    s = lax.axis_index("s")
    _zero_acc(nbuf, acc_sh, s)
    _edge_phase(c * 0, (s * 2 + c) * SL2, SL2 // CHUNK,
                src_hbm, dst_hbm, xl_hbm, xr_hbm, att_hbm,
                srcv, dstv, asrc, adst, xlr, xrr, prow, attv, acc_sh,
                sem1, sem2)
    for k in range(5):
        r0 = s * STRIPE + k * SUB
        pltpu.sync_copy(acc_sh.at[pl.ds(r0, SUB)], nbuf)
        pltpu.sync_copy(nbuf, out_hbm.at[pl.ds(c * N + r0, SUB)])


def _sc_l2(srcp, dstp, xl2, xr2, att2_flat):
    return pl.kernel(
        _scl2_body,
        out_type=jax.ShapeDtypeStruct((2 * N, AW), jnp.float32),
        mesh=_mesh,
        compiler_params=_params,
        scratch_types=[
            pltpu.VMEM((CHUNK,), jnp.int32),
            pltpu.VMEM((CHUNK,), jnp.int32),
            pltpu.VMEM((CHUNK,), jnp.int32),
            pltpu.VMEM((CHUNK,), jnp.int32),
            pltpu.VMEM((CHUNK, HID), jnp.float32),
            pltpu.VMEM((CHUNK, HID), jnp.float32),
            pltpu.VMEM((CHUNK, AW), jnp.float32),
            pltpu.VMEM((HID,), jnp.float32),
            pltpu.VMEM((SUB, AW), jnp.float32),
            pltpu.VMEM_SHARED((N, AW), jnp.float32),
            pltpu.SemaphoreType.DMA,
            pltpu.SemaphoreType.DMA,
        ],
    )(srcp, dstp, xl2, xr2, att2_flat)


def _tcc_body(pa_ref, pb_ref, b2_ref, batch_ref, w3_ref, b3_ref,
              w4_ref, b4_ref, out_ref, acc_ref):
    b = pl.program_id(0)

    @pl.when(b == 0)
    def _():
        acc_ref[...] = jnp.zeros((N_GRAPHS, HID), jnp.float32)

    num = pa_ref[:, 0:HID] + pb_ref[:, 0:HID]
    den = pa_ref[:, HID:HID + 1] + pb_ref[:, HID:HID + 1] + 1e-16
    h2 = jnp.maximum(num / den + b2_ref[...], 0.0)
    gid = batch_ref[0, :, :]
    iota = lax.broadcasted_iota(jnp.int32, (N_GRAPHS, BN), 0)
    oh = (iota == gid).astype(jnp.float32)
    acc_ref[...] += jnp.dot(oh, h2, preferred_element_type=jnp.float32)

    @pl.when(b == NB - 1)
    def _():
        g = jnp.maximum(jnp.dot(acc_ref[...], w3_ref[...],
                                preferred_element_type=jnp.float32) + b3_ref[...], 0.0)
        out_ref[...] = jnp.dot(g, w4_ref[...], preferred_element_type=jnp.float32) + b4_ref[...]


def _tc_c(out2_tab, b2r, batch3, W3, b3r, W4, b4r):
    return pl.pallas_call(
        _tcc_body,
        grid=(NB,),
        in_specs=[
            pl.BlockSpec((BN, AW), lambda b: (b, 0)),
            pl.BlockSpec((BN, AW), lambda b: (b + NB, 0)),
            pl.BlockSpec((1, HID), lambda b: (0, 0)),
            pl.BlockSpec((1, 1, BN), lambda b: (b, 0, 0)),
            pl.BlockSpec((HID, HID), lambda b: (0, 0)),
            pl.BlockSpec((1, HID), lambda b: (0, 0)),
            pl.BlockSpec((HID, 1), lambda b: (0, 0)),
            pl.BlockSpec((1, 1), lambda b: (0, 0)),
        ],
        out_specs=pl.BlockSpec((N_GRAPHS, 1), lambda b: (0, 0)),
        out_shape=jax.ShapeDtypeStruct((N_GRAPHS, 1), jnp.float32),
        scratch_shapes=[pltpu.VMEM((N_GRAPHS, HID), jnp.float32)],
    )(out2_tab, out2_tab, b2r, batch3, W3, b3r, W4, b4r)


def kernel(x, edge_index, batch, Wl1, bl1, Wr1, br1, att1, bias1,
           Wl2, bl2, Wr2, br2, att2, bias2, W3, b3, W4, b4):
    loops = jnp.arange(N, dtype=jnp.int32)
    src = jnp.concatenate([edge_index[0].astype(jnp.int32), loops,
                           jnp.zeros((E_PAD - E_TOT,), jnp.int32)])
    dst = jnp.concatenate([edge_index[1].astype(jnp.int32), loops,
                           jnp.zeros((E_PAD - E_TOT,), jnp.int32)])
    att1f = att1.reshape(4 * HID)

    xl4, xr4 = _tc_a(x, Wl1, bl1.reshape(1, 4 * HID), Wr1, br1.reshape(1, 4 * HID))
    xl_tab = xl4.reshape(4 * N, HID)
    xr_tab = xr4.reshape(4 * N, HID)
    o1a = _sc_l1(0, src, dst, xl_tab, xr_tab, att1f)
    att1f_dep = att1f + 0.0 * o1a[0, 0]
    o1b = _sc_l1(1, src, dst, xl_tab, xr_tab, att1f_dep)
    xl2, xr2 = _tc_b(o1a, o1b, bias1.reshape(4, HID), Wl2, bl2.reshape(1, HID),
                     Wr2, br2.reshape(1, HID))
    out2_tab = _sc_l2(src, dst, xl2, xr2, att2.reshape(HID))
    out = _tc_c(out2_tab, bias2.reshape(1, HID), batch.astype(jnp.int32).reshape(NB, 1, BN),
                W3, b3.reshape(1, HID), W4, b4.reshape(1, 1))
    return out


# speedup vs baseline: 38.9370x; 1.5944x over previous
import jax
import jax.numpy as jnp
from jax import lax
from jax.experimental import pallas as pl
from jax.experimental.pallas import tpu as pltpu
from jax.experimental.pallas import tpu_sc as plsc

N = 10000
E_TOT = 170000
D_IN = 128
HID = 64
N_GRAPHS = 64

CHUNK = 128
SL1 = 10752
SL2 = 5376
E_PAD = 172032
NB = 10
BN = N // NB
STRIPE = N // 16
SUB = 125
AW = 72

_mesh = plsc.VectorSubcoreMesh(core_axis_name="c", subcore_axis_name="s",
                               num_cores=2, num_subcores=16)
_params = pltpu.CompilerParams(use_tc_tiling_on_sc=False,
                               needs_layout_passes=False)

_GDN = lax.GatherDimensionNumbers(offset_dims=(), collapsed_slice_dims=(0,),
                                  start_index_map=(0,))


def _shuffle(vec, idx):
    return lax.gather(vec, idx[:, None], _GDN, (1,),
                      mode=lax.GatherScatterMode.PROMISE_IN_BOUNDS)


def _tca_body(x_ref, wl_ref, bl_ref, wr_ref, br_ref, xl_ref, xr_ref):
    xb = x_ref[...]
    xl = jnp.dot(xb, wl_ref[...], preferred_element_type=jnp.float32) + bl_ref[...]
    xr = jnp.dot(xb, wr_ref[...], preferred_element_type=jnp.float32) + br_ref[...]
    for h in range(4):
        xl_ref[h] = xl[:, h * HID:(h + 1) * HID]
        xr_ref[h] = xr[:, h * HID:(h + 1) * HID]


def _tc_a(x, Wl1, bl1r, Wr1, br1r):
    return pl.pallas_call(
        _tca_body,
        grid=(NB,),
        in_specs=[
            pl.BlockSpec((BN, D_IN), lambda b: (b, 0)),
            pl.BlockSpec((D_IN, 4 * HID), lambda b: (0, 0)),
            pl.BlockSpec((1, 4 * HID), lambda b: (0, 0)),
            pl.BlockSpec((D_IN, 4 * HID), lambda b: (0, 0)),
            pl.BlockSpec((1, 4 * HID), lambda b: (0, 0)),
        ],
        out_specs=[
            pl.BlockSpec((4, BN, HID), lambda b: (0, b, 0)),
            pl.BlockSpec((4, BN, HID), lambda b: (0, b, 0)),
        ],
        out_shape=[
            jax.ShapeDtypeStruct((4, N, HID), jnp.float32),
            jax.ShapeDtypeStruct((4, N, HID), jnp.float32),
        ],
    )(x, Wl1, bl1r, Wr1, br1r)


def _edge_phase(head, stripe, n_chunks, sl,
                src_hbm, dst_hbm, xl_hbm, xr_hbm, att_hbm,
                sidx, didx, dstv, adst, xlr, xrr, prow, attv, acc_sh,
                sems):
    pltpu.sync_copy(att_hbm.at[pl.ds(pl.multiple_of(head * HID, HID), HID)], attv)
    att_regs = [attv[pl.ds(k * 16, 16)] for k in range(4)]
    roff = jnp.full((16,), head * N, jnp.int32)
    lane = lax.iota(jnp.int32, 16)
    shift8 = (lane + 8) & 15
    sxl = [sems[0], sems[1]]
    sxr = [sems[2], sems[3]]

    st8 = pl.multiple_of(stripe, CHUNK)
    pltpu.sync_copy(src_hbm.at[pl.ds(st8, sl)], sidx)
    pltpu.sync_copy(dst_hbm.at[pl.ds(st8, sl)], didx)

    @plsc.parallel_loop(0, sl // 16)
    def _adj(i):
        sidx[pl.ds(i * 16, 16)] = sidx[pl.ds(i * 16, 16)] + roff

    def build_and_fire(ci, par):
        @plsc.parallel_loop(0, CHUNK // 16)
        def _bld(i):
            v = didx[pl.ds(ci * CHUNK + i * 16, 16)]
            dstv[par, pl.ds(i * 16, 16)] = v
            adst[par, pl.ds(i * 16, 16)] = v + roff

        pltpu.async_copy(xl_hbm.at[sidx.at[pl.ds(ci * CHUNK, CHUNK)]],
                         xlr.at[par], sxl[par])
        pltpu.async_copy(xr_hbm.at[adst.at[par]], xrr.at[par], sxr[par])

    build_and_fire(0, 0)
    build_and_fire(1, 1)

    def chunk_pair(ci2, _):
        for par in range(2):
            ci = ci2 * 2 + par
            base = stripe + ci * CHUNK
            pltpu.make_async_copy(xl_hbm.at[pl.ds(0, CHUNK)], xlr.at[par],
                                  sxl[par]).wait()
            pltpu.make_async_copy(xr_hbm.at[pl.ds(0, CHUNK)], xrr.at[par],
                                  sxr[par]).wait()

            @plsc.parallel_loop(0, CHUNK, unroll=4)
            def _edge(e):
                zls = []
                acc = jnp.zeros((16,), jnp.float32)
                for cb in range(4):
                    zl = xlr[par, e, pl.ds(cb * 16, 16)]
                    zr = xrr[par, e, pl.ds(cb * 16, 16)]
                    z = zl + zr
                    z = jnp.maximum(z, 0.2 * z)
                    acc = acc + z * att_regs[cb]
                    zls.append(zl)
                logit = jnp.sum(acc)
                vf = (base + e < E_TOT).astype(jnp.float32)
                p = jnp.exp(jnp.full((16,), logit, jnp.float32)) * vf
                for cb in range(3):
                    prow[e, pl.ds(cb * 16, 16)] = zls[cb] * p
                d3 = zls[3] * p
                prow[e, pl.ds(48, 16)] = d3
                tail = jnp.where(lane < 8, _shuffle(d3, shift8),
                                 jnp.where(lane == 8, p, 0.0))
                prow[e, pl.ds(56, 16)] = tail

            pltpu.sync_copy(prow, acc_sh.at[dstv.at[par]], add=True)

            @pl.when(ci + 2 < n_chunks)
            def _():
                build_and_fire(ci + 2, par)

        return 0

    lax.fori_loop(0, n_chunks // 2, chunk_pair, 0)
    plsc.subcore_barrier()


def _zero_acc(nbuf, acc_sh, s):
    @plsc.parallel_loop(0, SUB)
    def _zrow(r):
        for k in range(AW // 16):
            nbuf[r, pl.ds(k * 16, 16)] = jnp.zeros((16,), jnp.float32)
        nbuf[r, pl.ds(AW - 16, 16)] = jnp.zeros((16,), jnp.float32)

    for k in range(5):
        pltpu.sync_copy(nbuf, acc_sh.at[pl.ds(s * STRIPE + k * SUB, SUB)])
    plsc.subcore_barrier()


def _make_scl1_body(q):
    def body(src_hbm, dst_hbm, xl_hbm, xr_hbm, att_hbm, out_hbm,
             sidx, didx, dstv, adst, xlr, xrr, prow, attv, nbuf, obuf,
             acc_sh, sem1, sem2, sem3, sem4):
        c = lax.axis_index("c")
        s = lax.axis_index("s")
        _zero_acc(nbuf, acc_sh, s)
        head = 2 * q + c
        _edge_phase(head, s * SL1, SL1 // CHUNK, SL1,
                    src_hbm, dst_hbm, xl_hbm, xr_hbm, att_hbm,
                    sidx, didx, dstv, adst, xlr, xrr, prow, attv, acc_sh,
                    [sem1, sem2, sem3, sem4])
        idx8 = jnp.full((16,), 8, jnp.int32)
        for k in range(5):
            r0 = s * STRIPE + k * SUB
            pltpu.sync_copy(acc_sh.at[pl.ds(r0, SUB)], nbuf)

            @plsc.parallel_loop(0, SUB)
            def _nrow(r):
                svec = nbuf[r, pl.ds(56, 16)]
                inv = 1.0 / (svec + 1e-16)
                iv = _shuffle(inv, idx8)
                for cb in range(4):
                    obuf[r, pl.ds(cb * 16, 16)] = nbuf[r, pl.ds(cb * 16, 16)] * iv

            pltpu.sync_copy(obuf, out_hbm.at[pl.ds(head * N + r0, SUB)])

    return body


def _sc_l1(q, srcp, dstp, xl_tab, xr_tab, att_flat):
    return pl.kernel(
        _make_scl1_body(q),
        out_type=jax.ShapeDtypeStruct((4 * N, HID), jnp.float32),
        mesh=_mesh,
        compiler_params=_params,
        scratch_types=[
            pltpu.VMEM((SL1,), jnp.int32),
            pltpu.VMEM((SL1,), jnp.int32),
            pltpu.VMEM((2, CHUNK), jnp.int32),
            pltpu.VMEM((2, CHUNK), jnp.int32),
            pltpu.VMEM((2, CHUNK, HID), jnp.float32),
            pltpu.VMEM((2, CHUNK, HID), jnp.float32),
            pltpu.VMEM((CHUNK, AW), jnp.float32),
            pltpu.VMEM((HID,), jnp.float32),
            pltpu.VMEM((SUB, AW), jnp.float32),
            pltpu.VMEM((SUB, HID), jnp.float32),
            pltpu.VMEM_SHARED((N, AW), jnp.float32),
            pltpu.SemaphoreType.DMA,
            pltpu.SemaphoreType.DMA,
            pltpu.SemaphoreType.DMA,
            pltpu.SemaphoreType.DMA,
        ],
    )(srcp, dstp, xl_tab, xr_tab, att_flat)


def _tcb_body(h0_ref, h1_ref, h2_ref, h3_ref, b1_ref, wl_ref, bl_ref,
              wr_ref, br_ref, xl2_ref, xr2_ref):
    hs = [h0_ref, h1_ref, h2_ref, h3_ref]
    xl2 = bl_ref[...]
    xr2 = br_ref[...]
    for h in range(4):
        hh = jnp.maximum(hs[h][...] + b1_ref[h:h + 1, :], 0.0)
        xl2 = xl2 + jnp.dot(hh, wl_ref[pl.ds(h * HID, HID), :],
                            preferred_element_type=jnp.float32)
        xr2 = xr2 + jnp.dot(hh, wr_ref[pl.ds(h * HID, HID), :],
                            preferred_element_type=jnp.float32)
    xl2_ref[...] = xl2
    xr2_ref[...] = xr2


def _tc_b(o1a, o1b, b1r, Wl2, bl2r, Wr2, br2r):
    hspec = [
        pl.BlockSpec((BN, HID), lambda b: (0 * NB + b, 0)),
        pl.BlockSpec((BN, HID), lambda b: (1 * NB + b, 0)),
        pl.BlockSpec((BN, HID), lambda b: (2 * NB + b, 0)),
        pl.BlockSpec((BN, HID), lambda b: (3 * NB + b, 0)),
    ]
    return pl.pallas_call(
        _tcb_body,
        grid=(NB,),
        in_specs=hspec + [
            pl.BlockSpec((4, HID), lambda b: (0, 0)),
            pl.BlockSpec((4 * HID, HID), lambda b: (0, 0)),
            pl.BlockSpec((1, HID), lambda b: (0, 0)),
            pl.BlockSpec((4 * HID, HID), lambda b: (0, 0)),
            pl.BlockSpec((1, HID), lambda b: (0, 0)),
        ],
        out_specs=[
            pl.BlockSpec((BN, HID), lambda b: (b, 0)),
            pl.BlockSpec((BN, HID), lambda b: (b, 0)),
        ],
        out_shape=[
            jax.ShapeDtypeStruct((N, HID), jnp.float32),
            jax.ShapeDtypeStruct((N, HID), jnp.float32),
        ],
    )(o1a, o1a, o1b, o1b, b1r, Wl2, bl2r, Wr2, br2r)


def _scl2_body(src_hbm, dst_hbm, xl_hbm, xr_hbm, att_hbm, out_hbm,
               sidx, didx, dstv, adst, xlr, xrr, prow, attv, nbuf,
               acc_sh, sem1, sem2, sem3, sem4):
    c = lax.axis_index("c")
    s = lax.axis_index("s")
    _zero_acc(nbuf, acc_sh, s)
    _edge_phase(c * 0, (s * 2 + c) * SL2, SL2 // CHUNK, SL2,
                src_hbm, dst_hbm, xl_hbm, xr_hbm, att_hbm,
                sidx, didx, dstv, adst, xlr, xrr, prow, attv, acc_sh,
                [sem1, sem2, sem3, sem4])
    for k in range(5):
        r0 = s * STRIPE + k * SUB
        pltpu.sync_copy(acc_sh.at[pl.ds(r0, SUB)], nbuf)
        pltpu.sync_copy(nbuf, out_hbm.at[pl.ds(c * N + r0, SUB)])


def _sc_l2(srcp, dstp, xl2, xr2, att2_flat):
    return pl.kernel(
        _scl2_body,
        out_type=jax.ShapeDtypeStruct((2 * N, AW), jnp.float32),
        mesh=_mesh,
        compiler_params=_params,
        scratch_types=[
            pltpu.VMEM((SL2,), jnp.int32),
            pltpu.VMEM((SL2,), jnp.int32),
            pltpu.VMEM((2, CHUNK), jnp.int32),
            pltpu.VMEM((2, CHUNK), jnp.int32),
            pltpu.VMEM((2, CHUNK, HID), jnp.float32),
            pltpu.VMEM((2, CHUNK, HID), jnp.float32),
            pltpu.VMEM((CHUNK, AW), jnp.float32),
            pltpu.VMEM((HID,), jnp.float32),
            pltpu.VMEM((SUB, AW), jnp.float32),
            pltpu.VMEM_SHARED((N, AW), jnp.float32),
            pltpu.SemaphoreType.DMA,
            pltpu.SemaphoreType.DMA,
            pltpu.SemaphoreType.DMA,
            pltpu.SemaphoreType.DMA,
        ],
    )(srcp, dstp, xl2, xr2, att2_flat)


def _tcc_body(pa_ref, pb_ref, b2_ref, batch_ref, w3_ref, b3_ref,
              w4_ref, b4_ref, out_ref, acc_ref):
    b = pl.program_id(0)

    @pl.when(b == 0)
    def _():
        acc_ref[...] = jnp.zeros((N_GRAPHS, HID), jnp.float32)

    num = pa_ref[:, 0:HID] + pb_ref[:, 0:HID]
    den = pa_ref[:, HID:HID + 1] + pb_ref[:, HID:HID + 1] + 1e-16
    h2 = jnp.maximum(num / den + b2_ref[...], 0.0)
    gid = batch_ref[0, :, :]
    iota = lax.broadcasted_iota(jnp.int32, (N_GRAPHS, BN), 0)
    oh = (iota == gid).astype(jnp.float32)
    acc_ref[...] += jnp.dot(oh, h2, preferred_element_type=jnp.float32)

    @pl.when(b == NB - 1)
    def _():
        g = jnp.maximum(jnp.dot(acc_ref[...], w3_ref[...],
                                preferred_element_type=jnp.float32) + b3_ref[...], 0.0)
        out_ref[...] = jnp.dot(g, w4_ref[...], preferred_element_type=jnp.float32) + b4_ref[...]


def _tc_c(out2_tab, b2r, batch3, W3, b3r, W4, b4r):
    return pl.pallas_call(
        _tcc_body,
        grid=(NB,),
        in_specs=[
            pl.BlockSpec((BN, AW), lambda b: (b, 0)),
            pl.BlockSpec((BN, AW), lambda b: (b + NB, 0)),
            pl.BlockSpec((1, HID), lambda b: (0, 0)),
            pl.BlockSpec((1, 1, BN), lambda b: (b, 0, 0)),
            pl.BlockSpec((HID, HID), lambda b: (0, 0)),
            pl.BlockSpec((1, HID), lambda b: (0, 0)),
            pl.BlockSpec((HID, 1), lambda b: (0, 0)),
            pl.BlockSpec((1, 1), lambda b: (0, 0)),
        ],
        out_specs=pl.BlockSpec((N_GRAPHS, 1), lambda b: (0, 0)),
        out_shape=jax.ShapeDtypeStruct((N_GRAPHS, 1), jnp.float32),
        scratch_shapes=[pltpu.VMEM((N_GRAPHS, HID), jnp.float32)],
    )(out2_tab, out2_tab, b2r, batch3, W3, b3r, W4, b4r)


def kernel(x, edge_index, batch, Wl1, bl1, Wr1, br1, att1, bias1,
           Wl2, bl2, Wr2, br2, att2, bias2, W3, b3, W4, b4):
    loops = jnp.arange(N, dtype=jnp.int32)
    src = jnp.concatenate([edge_index[0].astype(jnp.int32), loops,
                           jnp.zeros((E_PAD - E_TOT,), jnp.int32)])
    dst = jnp.concatenate([edge_index[1].astype(jnp.int32), loops,
                           jnp.zeros((E_PAD - E_TOT,), jnp.int32)])
    att1f = att1.reshape(4 * HID)

    xl4, xr4 = _tc_a(x, Wl1, bl1.reshape(1, 4 * HID), Wr1, br1.reshape(1, 4 * HID))
    xl_tab = xl4.reshape(4 * N, HID)
    xr_tab = xr4.reshape(4 * N, HID)
    o1a = _sc_l1(0, src, dst, xl_tab, xr_tab, att1f)
    att1f_dep = att1f + 0.0 * o1a[0, 0]
    o1b = _sc_l1(1, src, dst, xl_tab, xr_tab, att1f_dep)
    xl2, xr2 = _tc_b(o1a, o1b, bias1.reshape(4, HID), Wl2, bl2.reshape(1, HID),
                     Wr2, br2.reshape(1, HID))
    out2_tab = _sc_l2(src, dst, xl2, xr2, att2.reshape(HID))
    out = _tc_c(out2_tab, bias2.reshape(1, HID), batch.astype(jnp.int32).reshape(NB, 1, BN),
                W3, b3.reshape(1, HID), W4, b4.reshape(1, 1))
    return out


# speedup vs baseline: 39.6421x; 1.0181x over previous
import jax
import jax.numpy as jnp
from jax import lax
from jax.experimental import pallas as pl
from jax.experimental.pallas import tpu as pltpu
from jax.experimental.pallas import tpu_sc as plsc

N = 10000
E_TOT = 170000
D_IN = 128
HID = 64
N_GRAPHS = 64

CHUNK = 128
SL1 = 10752
SL2 = 5376
E_PAD = 172032
NB = 10
BN = N // NB
STRIPE = N // 16
SUB = 125
AW = 72

_mesh = plsc.VectorSubcoreMesh(core_axis_name="c", subcore_axis_name="s",
                               num_cores=2, num_subcores=16)
_params = pltpu.CompilerParams(use_tc_tiling_on_sc=False,
                               needs_layout_passes=False)

_GDN = lax.GatherDimensionNumbers(offset_dims=(), collapsed_slice_dims=(0,),
                                  start_index_map=(0,))


def _shuffle(vec, idx):
    return lax.gather(vec, idx[:, None], _GDN, (1,),
                      mode=lax.GatherScatterMode.PROMISE_IN_BOUNDS)


def _tca_body(x_ref, wl_ref, bl_ref, wr_ref, br_ref, xl_ref, xr_ref):
    xb = x_ref[...]
    xl = jnp.dot(xb, wl_ref[...], preferred_element_type=jnp.float32) + bl_ref[...]
    xr = jnp.dot(xb, wr_ref[...], preferred_element_type=jnp.float32) + br_ref[...]
    for h in range(4):
        xl_ref[h] = xl[:, h * HID:(h + 1) * HID]
        xr_ref[h] = xr[:, h * HID:(h + 1) * HID]


def _tc_a(x, Wl1, bl1r, Wr1, br1r):
    return pl.pallas_call(
        _tca_body,
        grid=(NB,),
        in_specs=[
            pl.BlockSpec((BN, D_IN), lambda b: (b, 0)),
            pl.BlockSpec((D_IN, 4 * HID), lambda b: (0, 0)),
            pl.BlockSpec((1, 4 * HID), lambda b: (0, 0)),
            pl.BlockSpec((D_IN, 4 * HID), lambda b: (0, 0)),
            pl.BlockSpec((1, 4 * HID), lambda b: (0, 0)),
        ],
        out_specs=[
            pl.BlockSpec((4, BN, HID), lambda b: (0, b, 0)),
            pl.BlockSpec((4, BN, HID), lambda b: (0, b, 0)),
        ],
        out_shape=[
            jax.ShapeDtypeStruct((4, N, HID), jnp.float32),
            jax.ShapeDtypeStruct((4, N, HID), jnp.float32),
        ],
    )(x, Wl1, bl1r, Wr1, br1r)


def _edge_phase(head, stripe, n_chunks, sl,
                src_hbm, dst_hbm, xl_hbm, xr_hbm, att_hbm,
                sidx, didx, dstv, adst, xlr, xrr, prow, attv, acc_sh,
                sems):
    pltpu.sync_copy(att_hbm.at[pl.ds(pl.multiple_of(head * HID, HID), HID)], attv)
    att_regs = [attv[pl.ds(k * 16, 16)] for k in range(4)]
    roff = jnp.full((16,), head * N, jnp.int32)
    lane = lax.iota(jnp.int32, 16)
    shift8 = (lane + 8) & 15
    sxl = [sems[0], sems[1]]
    sxr = [sems[2], sems[3]]

    st8 = pl.multiple_of(stripe, CHUNK)
    pltpu.sync_copy(src_hbm.at[pl.ds(st8, sl)], sidx)
    pltpu.sync_copy(dst_hbm.at[pl.ds(st8, sl)], didx)

    @plsc.parallel_loop(0, sl // 16)
    def _adj(i):
        sidx[pl.ds(i * 16, 16)] = sidx[pl.ds(i * 16, 16)] + roff

    def build_and_fire(ci, par):
        @plsc.parallel_loop(0, CHUNK // 16)
        def _bld(i):
            v = didx[pl.ds(ci * CHUNK + i * 16, 16)]
            dstv[par, pl.ds(i * 16, 16)] = v
            adst[par, pl.ds(i * 16, 16)] = v + roff

        pltpu.async_copy(xl_hbm.at[sidx.at[pl.ds(ci * CHUNK, CHUNK)]],
                         xlr.at[par], sxl[par])
        pltpu.async_copy(xr_hbm.at[adst.at[par]], xrr.at[par], sxr[par])

    build_and_fire(0, 0)
    build_and_fire(1, 1)

    def chunk_pair(ci2, _):
        for par in range(2):
            ci = ci2 * 2 + par
            base = stripe + ci * CHUNK
            pltpu.make_async_copy(xl_hbm.at[pl.ds(0, CHUNK)], xlr.at[par],
                                  sxl[par]).wait()
            pltpu.make_async_copy(xr_hbm.at[pl.ds(0, CHUNK)], xrr.at[par],
                                  sxr[par]).wait()

            @plsc.parallel_loop(0, CHUNK, unroll=6)
            def _edge(e):
                zls = []
                acc = jnp.zeros((16,), jnp.float32)
                for cb in range(4):
                    zl = xlr[par, e, pl.ds(cb * 16, 16)]
                    zr = xrr[par, e, pl.ds(cb * 16, 16)]
                    z = zl + zr
                    z = jnp.maximum(z, 0.2 * z)
                    acc = acc + z * att_regs[cb]
                    zls.append(zl)
                logit = jnp.sum(acc)
                vf = (base + e < E_TOT).astype(jnp.float32)
                p = jnp.exp(jnp.full((16,), logit, jnp.float32)) * vf
                for cb in range(3):
                    prow[e, pl.ds(cb * 16, 16)] = zls[cb] * p
                d3 = zls[3] * p
                prow[e, pl.ds(48, 16)] = d3
                tail = jnp.where(lane < 8, _shuffle(d3, shift8),
                                 jnp.where(lane == 8, p, 0.0))
                prow[e, pl.ds(56, 16)] = tail

            pltpu.sync_copy(prow, acc_sh.at[dstv.at[par]], add=True)

            @pl.when(ci + 2 < n_chunks)
            def _():
                build_and_fire(ci + 2, par)

        return 0

    lax.fori_loop(0, n_chunks // 2, chunk_pair, 0)
    plsc.subcore_barrier()


def _zero_acc(nbuf, acc_sh, s):
    @plsc.parallel_loop(0, SUB)
    def _zrow(r):
        for k in range(AW // 16):
            nbuf[r, pl.ds(k * 16, 16)] = jnp.zeros((16,), jnp.float32)
        nbuf[r, pl.ds(AW - 16, 16)] = jnp.zeros((16,), jnp.float32)

    for k in range(5):
        pltpu.sync_copy(nbuf, acc_sh.at[pl.ds(s * STRIPE + k * SUB, SUB)])
    plsc.subcore_barrier()


def _make_scl1_body(q):
    def body(src_hbm, dst_hbm, xl_hbm, xr_hbm, att_hbm, out_hbm,
             sidx, didx, dstv, adst, xlr, xrr, prow, attv, nbuf, obuf,
             acc_sh, sem1, sem2, sem3, sem4):
        c = lax.axis_index("c")
        s = lax.axis_index("s")
        _zero_acc(nbuf, acc_sh, s)
        head = 2 * q + c
        _edge_phase(head, s * SL1, SL1 // CHUNK, SL1,
                    src_hbm, dst_hbm, xl_hbm, xr_hbm, att_hbm,
                    sidx, didx, dstv, adst, xlr, xrr, prow, attv, acc_sh,
                    [sem1, sem2, sem3, sem4])
        idx8 = jnp.full((16,), 8, jnp.int32)
        for k in range(5):
            r0 = s * STRIPE + k * SUB
            pltpu.sync_copy(acc_sh.at[pl.ds(r0, SUB)], nbuf)

            @plsc.parallel_loop(0, SUB)
            def _nrow(r):
                svec = nbuf[r, pl.ds(56, 16)]
                inv = 1.0 / (svec + 1e-16)
                iv = _shuffle(inv, idx8)
                for cb in range(4):
                    obuf[r, pl.ds(cb * 16, 16)] = nbuf[r, pl.ds(cb * 16, 16)] * iv

            pltpu.sync_copy(obuf, out_hbm.at[pl.ds(head * N + r0, SUB)])

    return body


def _sc_l1(q, srcp, dstp, xl_tab, xr_tab, att_flat):
    return pl.kernel(
        _make_scl1_body(q),
        out_type=jax.ShapeDtypeStruct((4 * N, HID), jnp.float32),
        mesh=_mesh,
        compiler_params=_params,
        scratch_types=[
            pltpu.VMEM((SL1,), jnp.int32),
            pltpu.VMEM((SL1,), jnp.int32),
            pltpu.VMEM((2, CHUNK), jnp.int32),
            pltpu.VMEM((2, CHUNK), jnp.int32),
            pltpu.VMEM((2, CHUNK, HID), jnp.float32),
            pltpu.VMEM((2, CHUNK, HID), jnp.float32),
            pltpu.VMEM((CHUNK, AW), jnp.float32),
            pltpu.VMEM((HID,), jnp.float32),
            pltpu.VMEM((SUB, AW), jnp.float32),
            pltpu.VMEM((SUB, HID), jnp.float32),
            pltpu.VMEM_SHARED((N, AW), jnp.float32),
            pltpu.SemaphoreType.DMA,
            pltpu.SemaphoreType.DMA,
            pltpu.SemaphoreType.DMA,
            pltpu.SemaphoreType.DMA,
        ],
    )(srcp, dstp, xl_tab, xr_tab, att_flat)


def _tcb_body(h0_ref, h1_ref, h2_ref, h3_ref, b1_ref, wl_ref, bl_ref,
              wr_ref, br_ref, xl2_ref, xr2_ref):
    hs = [h0_ref, h1_ref, h2_ref, h3_ref]
    xl2 = bl_ref[...]
    xr2 = br_ref[...]
    for h in range(4):
        hh = jnp.maximum(hs[h][...] + b1_ref[h:h + 1, :], 0.0)
        xl2 = xl2 + jnp.dot(hh, wl_ref[pl.ds(h * HID, HID), :],
                            preferred_element_type=jnp.float32)
        xr2 = xr2 + jnp.dot(hh, wr_ref[pl.ds(h * HID, HID), :],
                            preferred_element_type=jnp.float32)
    xl2_ref[...] = xl2
    xr2_ref[...] = xr2


def _tc_b(o1a, o1b, b1r, Wl2, bl2r, Wr2, br2r):
    hspec = [
        pl.BlockSpec((BN, HID), lambda b: (0 * NB + b, 0)),
        pl.BlockSpec((BN, HID), lambda b: (1 * NB + b, 0)),
        pl.BlockSpec((BN, HID), lambda b: (2 * NB + b, 0)),
        pl.BlockSpec((BN, HID), lambda b: (3 * NB + b, 0)),
    ]
    return pl.pallas_call(
        _tcb_body,
        grid=(NB,),
        in_specs=hspec + [
            pl.BlockSpec((4, HID), lambda b: (0, 0)),
            pl.BlockSpec((4 * HID, HID), lambda b: (0, 0)),
            pl.BlockSpec((1, HID), lambda b: (0, 0)),
            pl.BlockSpec((4 * HID, HID), lambda b: (0, 0)),
            pl.BlockSpec((1, HID), lambda b: (0, 0)),
        ],
        out_specs=[
            pl.BlockSpec((BN, HID), lambda b: (b, 0)),
            pl.BlockSpec((BN, HID), lambda b: (b, 0)),
        ],
        out_shape=[
            jax.ShapeDtypeStruct((N, HID), jnp.float32),
            jax.ShapeDtypeStruct((N, HID), jnp.float32),
        ],
    )(o1a, o1a, o1b, o1b, b1r, Wl2, bl2r, Wr2, br2r)


def _scl2_body(src_hbm, dst_hbm, xl_hbm, xr_hbm, att_hbm, out_hbm,
               sidx, didx, dstv, adst, xlr, xrr, prow, attv, nbuf,
               acc_sh, sem1, sem2, sem3, sem4):
    c = lax.axis_index("c")
    s = lax.axis_index("s")
    _zero_acc(nbuf, acc_sh, s)
    _edge_phase(c * 0, (s * 2 + c) * SL2, SL2 // CHUNK, SL2,
                src_hbm, dst_hbm, xl_hbm, xr_hbm, att_hbm,
                sidx, didx, dstv, adst, xlr, xrr, prow, attv, acc_sh,
                [sem1, sem2, sem3, sem4])
    for k in range(5):
        r0 = s * STRIPE + k * SUB
        pltpu.sync_copy(acc_sh.at[pl.ds(r0, SUB)], nbuf)
        pltpu.sync_copy(nbuf, out_hbm.at[pl.ds(c * N + r0, SUB)])


def _sc_l2(srcp, dstp, xl2, xr2, att2_flat):
    return pl.kernel(
        _scl2_body,
        out_type=jax.ShapeDtypeStruct((2 * N, AW), jnp.float32),
        mesh=_mesh,
        compiler_params=_params,
        scratch_types=[
            pltpu.VMEM((SL2,), jnp.int32),
            pltpu.VMEM((SL2,), jnp.int32),
            pltpu.VMEM((2, CHUNK), jnp.int32),
            pltpu.VMEM((2, CHUNK), jnp.int32),
            pltpu.VMEM((2, CHUNK, HID), jnp.float32),
            pltpu.VMEM((2, CHUNK, HID), jnp.float32),
            pltpu.VMEM((CHUNK, AW), jnp.float32),
            pltpu.VMEM((HID,), jnp.float32),
            pltpu.VMEM((SUB, AW), jnp.float32),
            pltpu.VMEM_SHARED((N, AW), jnp.float32),
            pltpu.SemaphoreType.DMA,
            pltpu.SemaphoreType.DMA,
            pltpu.SemaphoreType.DMA,
            pltpu.SemaphoreType.DMA,
        ],
    )(srcp, dstp, xl2, xr2, att2_flat)


def _tcc_body(pa_ref, pb_ref, b2_ref, batch_ref, w3_ref, b3_ref,
              w4_ref, b4_ref, out_ref, acc_ref):
    b = pl.program_id(0)

    @pl.when(b == 0)
    def _():
        acc_ref[...] = jnp.zeros((N_GRAPHS, HID), jnp.float32)

    num = pa_ref[:, 0:HID] + pb_ref[:, 0:HID]
    den = pa_ref[:, HID:HID + 1] + pb_ref[:, HID:HID + 1] + 1e-16
    h2 = jnp.maximum(num / den + b2_ref[...], 0.0)
    gid = batch_ref[0, :, :]
    iota = lax.broadcasted_iota(jnp.int32, (N_GRAPHS, BN), 0)
    oh = (iota == gid).astype(jnp.float32)
    acc_ref[...] += jnp.dot(oh, h2, preferred_element_type=jnp.float32)

    @pl.when(b == NB - 1)
    def _():
        g = jnp.maximum(jnp.dot(acc_ref[...], w3_ref[...],
                                preferred_element_type=jnp.float32) + b3_ref[...], 0.0)
        out_ref[...] = jnp.dot(g, w4_ref[...], preferred_element_type=jnp.float32) + b4_ref[...]


def _tc_c(out2_tab, b2r, batch3, W3, b3r, W4, b4r):
    return pl.pallas_call(
        _tcc_body,
        grid=(NB,),
        in_specs=[
            pl.BlockSpec((BN, AW), lambda b: (b, 0)),
            pl.BlockSpec((BN, AW), lambda b: (b + NB, 0)),
            pl.BlockSpec((1, HID), lambda b: (0, 0)),
            pl.BlockSpec((1, 1, BN), lambda b: (b, 0, 0)),
            pl.BlockSpec((HID, HID), lambda b: (0, 0)),
            pl.BlockSpec((1, HID), lambda b: (0, 0)),
            pl.BlockSpec((HID, 1), lambda b: (0, 0)),
            pl.BlockSpec((1, 1), lambda b: (0, 0)),
        ],
        out_specs=pl.BlockSpec((N_GRAPHS, 1), lambda b: (0, 0)),
        out_shape=jax.ShapeDtypeStruct((N_GRAPHS, 1), jnp.float32),
        scratch_shapes=[pltpu.VMEM((N_GRAPHS, HID), jnp.float32)],
    )(out2_tab, out2_tab, b2r, batch3, W3, b3r, W4, b4r)


def kernel(x, edge_index, batch, Wl1, bl1, Wr1, br1, att1, bias1,
           Wl2, bl2, Wr2, br2, att2, bias2, W3, b3, W4, b4):
    loops = jnp.arange(N, dtype=jnp.int32)
    src = jnp.concatenate([edge_index[0].astype(jnp.int32), loops,
                           jnp.zeros((E_PAD - E_TOT,), jnp.int32)])
    dst = jnp.concatenate([edge_index[1].astype(jnp.int32), loops,
                           jnp.zeros((E_PAD - E_TOT,), jnp.int32)])
    att1f = att1.reshape(4 * HID)

    xl4, xr4 = _tc_a(x, Wl1, bl1.reshape(1, 4 * HID), Wr1, br1.reshape(1, 4 * HID))
    xl_tab = xl4.reshape(4 * N, HID)
    xr_tab = xr4.reshape(4 * N, HID)
    o1a = _sc_l1(0, src, dst, xl_tab, xr_tab, att1f)
    att1f_dep = att1f + 0.0 * o1a[0, 0]
    o1b = _sc_l1(1, src, dst, xl_tab, xr_tab, att1f_dep)
    xl2, xr2 = _tc_b(o1a, o1b, bias1.reshape(4, HID), Wl2, bl2.reshape(1, HID),
                     Wr2, br2.reshape(1, HID))
    out2_tab = _sc_l2(src, dst, xl2, xr2, att2.reshape(HID))
    out = _tc_c(out2_tab, bias2.reshape(1, HID), batch.astype(jnp.int32).reshape(NB, 1, BN),
                W3, b3.reshape(1, HID), W4, b4.reshape(1, 1))
    return out
